# baseline (speedup 1.0000x reference)
import jax
import jax.numpy as jnp
from jax.experimental import pallas as pl
from jax.experimental.pallas import tpu as pltpu

_TILE = 1024
_STREAMS = 4


def _mlp_kernel(*refs):
    x_refs = refs[:_STREAMS]
    w1t_ref, b1_ref, w2t_ref, b2_ref, o_ref = refs[_STREAMS:]
    b1c = b1_ref[:][None, :].T
    b2c = b2_ref[:][None, :].T
    cols = []
    for x_ref in x_refs:
        hT = jax.lax.dot_general(
            w1t_ref[:], x_ref[:], (((1,), (1,)), ((), ())),
            preferred_element_type=jnp.float32,
        )
        hT = jnp.maximum(hT + b1c, 0.0)
        cols.append(
            jnp.dot(w2t_ref[:], hT, preferred_element_type=jnp.float32) + b2c
        )
    o_ref[:] = jnp.concatenate(cols, axis=1)


def _x_spec(j, last_block):
    return pl.BlockSpec(
        (_TILE, 128),
        lambda i: (jnp.minimum(_STREAMS * i + j, last_block), 0),
    )


def kernel(x, edge_index, W1, b1, W2, b2):
    n, in_ch = x.shape
    ncls = W2.shape[1]
    wide = _STREAMS * _TILE
    grid = (n + wide - 1) // wide
    outT = pl.pallas_call(
        _mlp_kernel,
        grid=(grid,),
        in_specs=[_x_spec(j, (n + _TILE - 1) // _TILE - 1)
                  for j in range(_STREAMS)] + [
            pl.BlockSpec(memory_space=pltpu.MemorySpace.VMEM),
            pl.BlockSpec(memory_space=pltpu.MemorySpace.VMEM),
            pl.BlockSpec(memory_space=pltpu.MemorySpace.VMEM),
            pl.BlockSpec(memory_space=pltpu.MemorySpace.VMEM),
        ],
        out_specs=pl.BlockSpec((ncls, wide), lambda i: (0, i)),
        out_shape=jax.ShapeDtypeStruct((ncls, n), jnp.float32),
        compiler_params=pltpu.CompilerParams(
            dimension_semantics=("parallel",),
        ),
    )(*([x] * _STREAMS), W1.T, b1, W2.T, b2)
    return outT.T


# speedup vs baseline: 1.0016x; 1.0016x over previous
import jax
import jax.numpy as jnp
from jax.experimental import pallas as pl
from jax.experimental.pallas import tpu as pltpu

_CHUNK = 1024


def _chunks(n):
    offs = list(range(0, n, _CHUNK))
    return [(o, min(_CHUNK, n - o)) for o in offs]


def _mlp_body(x_hbm, w1t_hbm, b1_hbm, w2t_hbm, b2_hbm, o_hbm,
              xv, ov, w1tv, b1v, w2tv, b2v, lsem, wsem, ssem):
    n = x_hbm.shape[0]
    chunks = _chunks(n)

    wcopies = (
        pltpu.make_async_copy(w1t_hbm, w1tv, wsem.at[0]),
        pltpu.make_async_copy(b1_hbm, b1v, wsem.at[1]),
        pltpu.make_async_copy(w2t_hbm, w2tv, wsem.at[2]),
        pltpu.make_async_copy(b2_hbm, b2v, wsem.at[3]),
    )
    for c in wcopies:
        c.start()

    lcopies = []
    for k, (o, s) in enumerate(chunks):
        c = pltpu.make_async_copy(
            x_hbm.at[pl.ds(o, s), :], xv.at[pl.ds(o, s), :], lsem.at[k]
        )
        c.start()
        lcopies.append(c)

    for c in wcopies:
        c.wait()
    b1c = b1v[:][None, :].T
    b2c = b2v[:][None, :].T

    scopies = []
    for k, (o, s) in enumerate(chunks):
        lcopies[k].wait()
        hT = jax.lax.dot_general(
            w1tv[:], xv[pl.ds(o, s), :], (((1,), (1,)), ((), ())),
            preferred_element_type=jnp.float32,
        )
        hT = jnp.maximum(hT + b1c, 0.0)
        ov[:, pl.ds(o, s)] = (
            jnp.dot(w2tv[:], hT, preferred_element_type=jnp.float32) + b2c
        )
        c = pltpu.make_async_copy(
            ov.at[:, pl.ds(o, s)], o_hbm.at[:, pl.ds(o, s)], ssem.at[k]
        )
        c.start()
        scopies.append(c)

    for c in scopies:
        c.wait()


def kernel(x, edge_index, W1, b1, W2, b2):
    n, in_ch = x.shape
    hid = W1.shape[1]
    ncls = W2.shape[1]
    nchunks = len(_chunks(n))
    outT = pl.pallas_call(
        _mlp_body,
        in_specs=[pl.BlockSpec(memory_space=pltpu.MemorySpace.HBM)] * 5,
        out_specs=pl.BlockSpec(memory_space=pltpu.MemorySpace.HBM),
        out_shape=jax.ShapeDtypeStruct((ncls, n), jnp.float32),
        scratch_shapes=[
            pltpu.VMEM((n, in_ch), jnp.float32),
            pltpu.VMEM((ncls, n), jnp.float32),
            pltpu.VMEM((hid, in_ch), jnp.float32),
            pltpu.VMEM((hid,), jnp.float32),
            pltpu.VMEM((ncls, hid), jnp.float32),
            pltpu.VMEM((ncls,), jnp.float32),
            pltpu.SemaphoreType.DMA((nchunks,)),
            pltpu.SemaphoreType.DMA((4,)),
            pltpu.SemaphoreType.DMA((nchunks,)),
        ],
    )(x, W1.T, b1, W2.T, b2)
    return outT.T


# speedup vs baseline: 1.5163x; 1.5138x over previous
import jax
import jax.numpy as jnp
from jax.experimental import pallas as pl
from jax.experimental.pallas import tpu as pltpu

_TILE = 5120


def _mlp_kernel(x_ref, w1t_ref, b1_ref, w2t_ref, b2_ref, o_ref):
    hT = jax.lax.dot_general(
        w1t_ref[:], x_ref[:], (((1,), (1,)), ((), ())),
        preferred_element_type=jnp.float32,
    )
    b1c = b1_ref[:][None, :].T
    hT = jnp.maximum(hT + b1c, 0.0)
    oT = jnp.dot(w2t_ref[:], hT, preferred_element_type=jnp.float32)
    b2c = b2_ref[:][None, :].T
    o_ref[:] = oT + b2c


def kernel(x, edge_index, W1, b1, W2, b2):
    n, in_ch = x.shape
    hid = W1.shape[1]
    ncls = W2.shape[1]
    grid = (n + _TILE - 1) // _TILE
    outT = pl.pallas_call(
        _mlp_kernel,
        grid=(grid,),
        in_specs=[
            pl.BlockSpec((_TILE, in_ch), lambda i: (i, 0)),
            pl.BlockSpec(memory_space=pltpu.MemorySpace.VMEM),
            pl.BlockSpec(memory_space=pltpu.MemorySpace.VMEM),
            pl.BlockSpec(memory_space=pltpu.MemorySpace.VMEM),
            pl.BlockSpec(memory_space=pltpu.MemorySpace.VMEM),
        ],
        out_specs=pl.BlockSpec((ncls, _TILE), lambda i: (0, i)),
        out_shape=jax.ShapeDtypeStruct((ncls, n), jnp.float32),
        compiler_params=pltpu.CompilerParams(
            dimension_semantics=("arbitrary",),
        ),
    )(x, W1.T, b1, W2.T, b2)
    return outT.T
